# baseline (speedup 1.0000x reference)
import jax
import jax.numpy as jnp
from jax.experimental import pallas as pl
from jax.experimental.pallas import tpu as pltpu

_B = 32
_H = 384
_W = 384
_N = _B * _H * _W
_BB = 2


def _tc_body(sal_ref, fix_ref, out_ref, acc_ref):
    i = pl.program_id(0)
    ni = pl.num_programs(0)

    @pl.when(i == 0)
    def _init():
        acc_ref[...] = jnp.zeros_like(acc_ref)

    s = sal_ref[...]
    f = fix_ref[...]
    m = f > 0.1
    r = _BB * _H // 8
    sb = s.reshape(r, 8, _W)
    fb = jnp.where(m, s, 0.0).reshape(r, 8, _W)
    cb = m.astype(jnp.float32).reshape(r, 8, _W)
    acc_ref[0] += jnp.sum(sb, axis=0)
    acc_ref[1] += jnp.sum(sb * sb, axis=0)
    acc_ref[2] += jnp.sum(fb, axis=0)
    acc_ref[3] += jnp.sum(cb, axis=0)

    @pl.when(i == ni - 1)
    def _fin():
        s1 = jnp.sum(acc_ref[0])
        s2 = jnp.sum(acc_ref[1])
        ms = jnp.sum(acc_ref[2])
        cnt = jnp.sum(acc_ref[3])
        n = jnp.float32(_N)
        mean = s1 / n
        var = (s2 - s1 * s1 / n) / (n - 1.0)
        std = jnp.sqrt(var)
        out_ref[0] = (ms - cnt * mean) / (std * cnt)


def kernel(sal_map, fix):
    out = pl.pallas_call(
        _tc_body,
        grid=(_B // _BB,),
        in_specs=[
            pl.BlockSpec((_BB, 1, _H, _W), lambda i: (i, 0, 0, 0)),
            pl.BlockSpec((_BB, 1, _H, _W), lambda i: (i, 0, 0, 0)),
        ],
        out_specs=pl.BlockSpec(memory_space=pltpu.SMEM),
        out_shape=jax.ShapeDtypeStruct((1,), jnp.float32),
        scratch_shapes=[pltpu.VMEM((4, 8, _W), jnp.float32)],
    )(sal_map, fix)
    return out[0]


# speedup vs baseline: 1.3353x; 1.3353x over previous
import jax
import jax.numpy as jnp
from jax.experimental import pallas as pl
from jax.experimental.pallas import tpu as pltpu

_B = 32
_H = 384
_W = 384
_N = _B * _H * _W
_BB = 8


def _tc_body(sal_ref, fix_ref, out_ref, acc_ref):
    i = pl.program_id(0)
    ni = pl.num_programs(0)

    @pl.when(i == 0)
    def _init():
        acc_ref[...] = jnp.zeros_like(acc_ref)

    s = sal_ref[...]
    f = fix_ref[...]
    m = f > 0.1
    r = _BB * _H // 8
    sb = s.reshape(r, 8, _W)
    fb = jnp.where(m, s, 0.0).reshape(r, 8, _W)
    cb = m.astype(jnp.float32).reshape(r, 8, _W)
    acc_ref[0] += jnp.sum(sb, axis=0)
    acc_ref[1] += jnp.sum(sb * sb, axis=0)
    acc_ref[2] += jnp.sum(fb, axis=0)
    acc_ref[3] += jnp.sum(cb, axis=0)

    @pl.when(i == ni - 1)
    def _fin():
        s1 = jnp.sum(acc_ref[0])
        s2 = jnp.sum(acc_ref[1])
        ms = jnp.sum(acc_ref[2])
        cnt = jnp.sum(acc_ref[3])
        n = jnp.float32(_N)
        mean = s1 / n
        var = (s2 - s1 * s1 / n) / (n - 1.0)
        std = jnp.sqrt(var)
        out_ref[0] = (ms - cnt * mean) / (std * cnt)


def kernel(sal_map, fix):
    out = pl.pallas_call(
        _tc_body,
        grid=(_B // _BB,),
        in_specs=[
            pl.BlockSpec((_BB, 1, _H, _W), lambda i: (i, 0, 0, 0)),
            pl.BlockSpec((_BB, 1, _H, _W), lambda i: (i, 0, 0, 0)),
        ],
        out_specs=pl.BlockSpec(memory_space=pltpu.SMEM),
        out_shape=jax.ShapeDtypeStruct((1,), jnp.float32),
        scratch_shapes=[pltpu.VMEM((4, 8, _W), jnp.float32)],
    )(sal_map, fix)
    return out[0]
